# baseline (speedup 1.0000x reference)
import functools

import jax
import jax.numpy as jnp
from jax import lax
from jax.experimental import pallas as pl
from jax.experimental.pallas import tpu as pltpu
from jax.experimental.pallas import tpu_sc as plsc

_D = 256
_MW = 50


def _build_sc_gather(N, T):
    info = plsc.get_sparse_core_info()
    NC, NS, L = info.num_cores, info.num_subcores, info.num_lanes
    NW = NC * NS
    NPW = N // NW
    RPW = NPW * _MW
    CH = 128
    NCH = RPW // CH
    ZROW = T

    mesh = plsc.VectorSubcoreMesh(core_axis_name="c", subcore_axis_name="s")

    @functools.partial(
        pl.kernel,
        mesh=mesh,
        out_type=jax.ShapeDtypeStruct((N * _MW, _D), jnp.float32),
        scratch_types=[
            pltpu.VMEM((NPW,), jnp.int32),
            pltpu.VMEM((NPW,), jnp.int32),
            pltpu.VMEM((RPW + 2 * L,), jnp.int32),
            pltpu.VMEM((3, CH, _D), jnp.float32),
            pltpu.VMEM_SHARED((8, _D), jnp.float32),
            pltpu.SemaphoreType.DMA,
            pltpu.SemaphoreType.DMA,
        ],
    )
    def sc_gather(starts_hbm, ends_hbm, table_hbm, out_hbm,
                  starts_v, ends_v, idx_v, buf, spdummy, gsem, ssem):
        wid = lax.axis_index("s") * NC + lax.axis_index("c")
        nbase = pl.multiple_of(wid * NPW, NPW)
        pltpu.sync_copy(starts_hbm.at[pl.ds(nbase, NPW)], starts_v)
        pltpu.sync_copy(ends_hbm.at[pl.ds(nbase, NPW)], ends_v)

        lane = lax.iota(jnp.int32, L)
        njc = (_MW + L - 1) // L

        def group_body(g, _):
            goff = pl.multiple_of(g * L, L)
            sv = starts_v[pl.ds(goff, L)]
            ev = ends_v[pl.ds(goff, L)]
            gbase = g * (L * _MW)
            for i in range(L):
                s = sv[i]
                e = ev[i]
                meta = e < 0
                eff = jnp.where(meta, T - 1, s)
                cnt = jnp.where(meta, 1, e - s + 1)
                for jc in range(njc):
                    jv = jc * L + lane
                    idx = jnp.where(jv < cnt, eff + jv, ZROW)
                    idx_v[pl.ds(gbase + (i * _MW + jc * L), L)] = idx
            return 0

        lax.fori_loop(0, NPW // L, group_body, 0)

        rbase = wid * RPW
        NBUF = 3

        def gsrc(c):
            return table_hbm.at[idx_v.at[pl.ds(pl.multiple_of(c * CH, CH), CH)]]

        def odst(c):
            return out_hbm.at[pl.ds(pl.multiple_of(rbase + c * CH, CH), CH)]

        for b in range(NBUF):
            pltpu.async_copy(gsrc(b), buf.at[b], gsem)

        def slot_body(cc, _):
            c0 = cc * NBUF
            for b in range(NBUF):
                c = c0 + b

                @pl.when(c < NCH)
                def _():
                    pltpu.make_async_copy(gsrc(c), buf.at[b], gsem).wait()
                    pltpu.async_copy(buf.at[b], odst(c), ssem)
                    pltpu.make_async_copy(buf.at[b], odst(c), ssem).wait()

                    @pl.when(c + NBUF < NCH)
                    def _():
                        pltpu.async_copy(gsrc(c + NBUF), buf.at[b], gsem)

            return 0

        lax.fori_loop(0, (NCH + NBUF - 1) // NBUF, slot_body, 0)

    return sc_gather


def kernel(batched_nodes, batched_bi_lstm_outputs):
    nodes0 = batched_nodes[0]
    H = batched_bi_lstm_outputs[0]
    N = nodes0.shape[0]
    T = H.shape[0]
    starts = nodes0[:, 0]
    ends = nodes0[:, 1]
    table = jnp.concatenate([H, jnp.zeros((1, _D), H.dtype)], axis=0)
    out = _build_sc_gather(N, T)(starts, ends, table)
    return out.reshape(1, N, _MW, _D)


# speedup vs baseline: 8.5614x; 8.5614x over previous
import functools

import jax
import jax.numpy as jnp
from jax import lax
from jax.experimental import pallas as pl
from jax.experimental.pallas import tpu as pltpu
from jax.experimental.pallas import tpu_sc as plsc

_D = 256
_MW = 50


def _build_sc_copy(N, T):
    info = plsc.get_sparse_core_info()
    NC, NS, L = info.num_cores, info.num_subcores, info.num_lanes
    NW = NC * NS
    NPW = N // NW
    RPW = NPW * _MW

    mesh = plsc.VectorSubcoreMesh(core_axis_name="c", subcore_axis_name="s")

    @functools.partial(
        pl.kernel,
        mesh=mesh,
        compiler_params=pltpu.CompilerParams(use_tc_tiling_on_sc=False),
        out_type=jax.ShapeDtypeStruct((N * _MW, _D), jnp.float32),
        scratch_types=[
            pltpu.VMEM((NPW + L,), jnp.int32),
            pltpu.VMEM((NPW + L,), jnp.int32),
            pltpu.VMEM_SHARED((T, _D), jnp.float32),
            pltpu.VMEM_SHARED((_MW, _D), jnp.float32),
            pltpu.SemaphoreType.DMA,
        ],
    )
    def sc_copy(starts_hbm, ends_hbm, table_hbm, zeros_hbm, out_hbm,
                starts_v, ends_v, sp_table, sp_zeros, dsem):
        cid = lax.axis_index("c")
        sid = lax.axis_index("s")
        wid = cid * NS + sid
        nbase = pl.multiple_of(wid * NPW, NPW)
        pltpu.sync_copy(starts_hbm.at[pl.ds(nbase, NPW)],
                        starts_v.at[pl.ds(0, NPW)])
        pltpu.sync_copy(ends_hbm.at[pl.ds(nbase, NPW)],
                        ends_v.at[pl.ds(0, NPW)])

        pltpu.sync_copy(table_hbm, sp_table)
        pltpu.sync_copy(zeros_hbm, sp_zeros)

        rbase = wid * RPW

        def mk_branch(k, eff, obase):
            def br():
                pltpu.async_copy(
                    sp_table.at[pl.ds(eff, k)],
                    out_hbm.at[pl.ds(obase, k)], dsem)
                if k < _MW:
                    pltpu.async_copy(
                        sp_zeros.at[pl.ds(0, _MW - k)],
                        out_hbm.at[pl.ds(obase + k, _MW - k)], dsem)
            return br

        def node_body(n, _):
            sv = starts_v[pl.ds(n, L)]
            ev = ends_v[pl.ds(n, L)]
            s = sv[0]
            e = ev[0]
            meta = e < 0
            eff = jnp.where(meta, T - 1, s)
            cnt = jnp.where(meta, 1, e - s + 1)
            obase = rbase + n * _MW
            lax.switch(cnt - 1,
                       [mk_branch(k, eff, obase)
                        for k in range(1, _MW + 1)])
            return 0

        lax.fori_loop(0, NPW, node_body, 0)

        myout = out_hbm.at[pl.ds(pl.multiple_of(rbase, RPW), RPW)]
        pltpu.make_async_copy(myout, myout, dsem).wait()

    return sc_copy


def kernel(batched_nodes, batched_bi_lstm_outputs):
    nodes0 = batched_nodes[0]
    H = batched_bi_lstm_outputs[0]
    N = nodes0.shape[0]
    T = H.shape[0]
    starts = nodes0[:, 0]
    ends = nodes0[:, 1]
    zeros = jnp.zeros((_MW, _D), H.dtype)
    out = _build_sc_copy(N, T)(starts, ends, H, zeros)
    return out.reshape(1, N, _MW, _D)


# speedup vs baseline: 11.3336x; 1.3238x over previous
import functools

import jax
import jax.numpy as jnp
from jax import lax
from jax.experimental import pallas as pl
from jax.experimental.pallas import tpu as pltpu
from jax.experimental.pallas import tpu_sc as plsc

_D = 256
_MW = 50
_ZR = 40


def _mesh_info():
    info = plsc.get_sparse_core_info()
    return info.num_cores, info.num_subcores, info.num_lanes


def _build_mixed_tiles(N, T):
    NC, NS, L = _mesh_info()
    NW = NC * NS
    NPW = N // NW

    mesh = plsc.VectorSubcoreMesh(core_axis_name="c", subcore_axis_name="s")

    @functools.partial(
        pl.kernel,
        mesh=mesh,
        compiler_params=pltpu.CompilerParams(use_tc_tiling_on_sc=False),
        out_type=jax.ShapeDtypeStruct((N * 8, _D), jnp.float32),
        scratch_types=[
            pltpu.VMEM((NPW + L,), jnp.int32),
            pltpu.VMEM((NPW + L,), jnp.int32),
            pltpu.VMEM_SHARED((T, _D), jnp.float32),
            pltpu.VMEM_SHARED((8, _D), jnp.float32),
            pltpu.SemaphoreType.DMA,
        ],
    )
    def mk_mixed(starts_hbm, ends_hbm, table_hbm, zeros_hbm, mixed_hbm,
                 starts_v, ends_v, sp_table, sp_zeros, dsem):
        cid = lax.axis_index("c")
        sid = lax.axis_index("s")
        wid = cid * NS + sid
        nbase = pl.multiple_of(wid * NPW, NPW)
        pltpu.sync_copy(starts_hbm.at[pl.ds(nbase, NPW)],
                        starts_v.at[pl.ds(0, NPW)])
        pltpu.sync_copy(ends_hbm.at[pl.ds(nbase, NPW)],
                        ends_v.at[pl.ds(0, NPW)])

        TROWS = T // NS
        soff = pl.multiple_of(sid * TROWS, TROWS)
        pltpu.sync_copy(table_hbm.at[pl.ds(soff, TROWS)],
                        sp_table.at[pl.ds(soff, TROWS)])

        @pl.when(sid == 0)
        def _():
            pltpu.sync_copy(zeros_hbm.at[pl.ds(0, 8)], sp_zeros)

        plsc.subcore_barrier()

        def mk_branch(k, eff, mbase):
            t, r = divmod(k, 8)

            def br():
                if k >= 49:
                    nd = k - 48
                    pltpu.async_copy(sp_table.at[pl.ds(eff + 48, nd)],
                                     mixed_hbm.at[pl.ds(mbase, nd)], dsem)
                    if nd < 2:
                        pltpu.async_copy(
                            sp_zeros.at[pl.ds(0, 2 - nd)],
                            mixed_hbm.at[pl.ds(mbase + nd, 2 - nd)], dsem)
                elif r != 0:
                    pltpu.async_copy(sp_table.at[pl.ds(eff + 8 * t, r)],
                                     mixed_hbm.at[pl.ds(mbase, r)], dsem)
                    pltpu.async_copy(sp_zeros.at[pl.ds(0, 8 - r)],
                                     mixed_hbm.at[pl.ds(mbase + r, 8 - r)],
                                     dsem)
            return br

        def node_body(n, rows):
            sv = starts_v[pl.ds(n, L)]
            ev = ends_v[pl.ds(n, L)]
            s = sv[0]
            e = ev[0]
            meta = e < 0
            eff = jnp.where(meta, T - 1, s)
            cnt = jnp.where(meta, 1, e - s + 1)
            mbase = (nbase + n) * 8
            lax.switch(cnt - 1,
                       [mk_branch(k, eff, mbase) for k in range(1, _MW + 1)])
            r8 = cnt % 8
            add = jnp.where(cnt >= 49, 2, jnp.where(r8 == 0, 0, 8))
            return rows + add

        rows = lax.fori_loop(0, NPW, node_body, jnp.int32(0))

        unit = mixed_hbm.at[pl.ds(0, 1)]

        def drain(i, _):
            pltpu.make_async_copy(unit, unit, dsem).wait()
            return 0

        lax.fori_loop(0, rows, drain, 0)

    return mk_mixed


def _build_main(N, T):
    NC, NS, L = _mesh_info()
    NW = NC * NS
    NPW = N // NW
    CHN = 32
    NCH = NPW // CHN

    mesh = plsc.VectorSubcoreMesh(core_axis_name="c", subcore_axis_name="s")

    @functools.partial(
        pl.kernel,
        mesh=mesh,
        out_type=jax.ShapeDtypeStruct((N, _MW, _D), jnp.float32),
        scratch_types=[
            pltpu.VMEM((NPW + L,), jnp.int32),
            pltpu.VMEM((NPW + L,), jnp.int32),
            pltpu.VMEM_SHARED((T, _D), jnp.float32),
            pltpu.VMEM_SHARED((_ZR, _D), jnp.float32),
            pltpu.VMEM_SHARED((NS * CHN * 8, _D), jnp.float32),
            pltpu.SemaphoreType.DMA,
            pltpu.SemaphoreType.DMA,
        ],
    )
    def main(starts_hbm, ends_hbm, shifted_hbm, zeros_hbm, mixed_hbm,
             out_hbm, starts_v, ends_v, sp_shift, sp_zeros, sp_mixed,
             dsem, dsem2):
        cid = lax.axis_index("c")
        sid = lax.axis_index("s")
        wid = cid * NS + sid
        nbase = pl.multiple_of(wid * NPW, NPW)
        pltpu.sync_copy(starts_hbm.at[pl.ds(nbase, NPW)],
                        starts_v.at[pl.ds(0, NPW)])
        pltpu.sync_copy(ends_hbm.at[pl.ds(nbase, NPW)],
                        ends_v.at[pl.ds(0, NPW)])

        @pl.when(sid == 0)
        def _():
            pltpu.sync_copy(zeros_hbm, sp_zeros)

        plsc.subcore_barrier()

        sbase = sid * CHN * 8
        TROWS = T // NS
        soff = pl.multiple_of(sid * TROWS, TROWS)

        def mk_branch1(k, n, j):
            t, r = divmod(k, 8)

            def br():
                if k >= 49:
                    pltpu.async_copy(sp_mixed.at[pl.ds(sbase + j * 8, 2)],
                                     out_hbm.at[n, pl.ds(48, 2)], dsem)
                    return
                if r != 0:
                    pltpu.async_copy(sp_mixed.at[pl.ds(sbase + j * 8, 8)],
                                     out_hbm.at[n, pl.ds(8 * t, 8)], dsem)
                z0 = 8 * t + (8 if r != 0 else 0)
                if z0 < 48:
                    pltpu.async_copy(sp_zeros.at[pl.ds(0, 48 - z0)],
                                     out_hbm.at[n, pl.ds(z0, 48 - z0)], dsem)
                pltpu.async_copy(sp_zeros.at[pl.ds(0, 2)],
                                 out_hbm.at[n, pl.ds(48, 2)], dsem)
            return br

        def chunk_body(c, rows1):
            cb = pl.multiple_of(c * CHN, CHN)
            pltpu.sync_copy(
                mixed_hbm.at[pl.ds(pl.multiple_of((nbase + cb) * 8, 8 * CHN),
                                   8 * CHN)],
                sp_mixed.at[pl.ds(pl.multiple_of(sbase, 8 * CHN), 8 * CHN)])

            def node_body(j, rws):
                sv = starts_v[pl.ds(cb + j, L)]
                ev = ends_v[pl.ds(cb + j, L)]
                s = sv[0]
                e = ev[0]
                meta = e < 0
                cnt = jnp.where(meta, 1, e - s + 1)
                lax.switch(cnt - 1,
                           [mk_branch1(k, nbase + cb + j, j)
                            for k in range(1, _MW + 1)])
                return rws + (_MW - 8 * (cnt // 8))

            return lax.fori_loop(0, CHN, node_body, rows1)

        rows1 = lax.fori_loop(0, NCH, chunk_body, jnp.int32(0))

        def mk_branch2(k, eff, n, rho):
            q = min(k // 8, 6)

            def br():
                if q > 0:
                    effa = pl.multiple_of(eff - rho, 8)
                    pltpu.async_copy(sp_shift.at[pl.ds(effa, 8 * q)],
                                     out_hbm.at[n, pl.ds(0, 8 * q)], dsem2)
            return br

        unit = out_hbm.at[0, pl.ds(0, 1)]

        def pass_body(rho, _):
            plsc.subcore_barrier()
            pltpu.sync_copy(shifted_hbm.at[rho, pl.ds(soff, TROWS)],
                            sp_shift.at[pl.ds(soff, TROWS)])
            plsc.subcore_barrier()

            def node_body(n, rows2):
                sv = starts_v[pl.ds(n, L)]
                ev = ends_v[pl.ds(n, L)]
                s = sv[0]
                e = ev[0]
                meta = e < 0
                eff = jnp.where(meta, T - 1, s)
                cnt = jnp.where(meta, 1, e - s + 1)
                mine = lax.rem(eff, 8) == rho

                @pl.when(mine)
                def _():
                    lax.switch(cnt - 1,
                               [mk_branch2(k, eff, nbase + n, rho)
                                for k in range(1, _MW + 1)])

                q = jnp.minimum(cnt // 8, 6)
                return rows2 + jnp.where(mine, 8 * q, 0)

            rows2 = lax.fori_loop(0, NPW, node_body, jnp.int32(0))

            def drain2(i, _):
                pltpu.make_async_copy(unit, unit, dsem2).wait()
                return 0

            lax.fori_loop(0, rows2, drain2, 0)
            return 0

        lax.fori_loop(0, 8, pass_body, 0)

        def drain1(i, _):
            pltpu.make_async_copy(unit, unit, dsem).wait()
            return 0

        lax.fori_loop(0, rows1, drain1, 0)

    return main


def kernel(batched_nodes, batched_bi_lstm_outputs):
    nodes0 = batched_nodes[0]
    H = batched_bi_lstm_outputs[0]
    N = nodes0.shape[0]
    T = H.shape[0]
    starts = nodes0[:, 0]
    ends = nodes0[:, 1]
    zeros = jnp.zeros((_ZR, _D), H.dtype)
    shifted = jnp.stack([jnp.roll(H, -r, axis=0) for r in range(8)])
    mixed = _build_mixed_tiles(N, T)(starts, ends, H, zeros)
    out = _build_main(N, T)(starts, ends, shifted, zeros, mixed)
    return out.reshape(1, N, _MW, _D)


# speedup vs baseline: 11.4601x; 1.0112x over previous
import functools

import jax
import jax.numpy as jnp
from jax import lax
from jax.experimental import pallas as pl
from jax.experimental.pallas import tpu as pltpu
from jax.experimental.pallas import tpu_sc as plsc

_D = 256
_MW = 50
_ZR = 40


def _mesh_info():
    info = plsc.get_sparse_core_info()
    return info.num_cores, info.num_subcores, info.num_lanes


def _build_mixed_tiles(N, T):
    NC, NS, L = _mesh_info()
    NW = NC * NS
    NPW = N // NW

    mesh = plsc.VectorSubcoreMesh(core_axis_name="c", subcore_axis_name="s")

    @functools.partial(
        pl.kernel,
        mesh=mesh,
        compiler_params=pltpu.CompilerParams(use_tc_tiling_on_sc=False),
        out_type=jax.ShapeDtypeStruct((N * 8, _D), jnp.float32),
        scratch_types=[
            pltpu.VMEM((NPW + L,), jnp.int32),
            pltpu.VMEM((NPW + L,), jnp.int32),
            pltpu.VMEM_SHARED((T, _D), jnp.float32),
            pltpu.VMEM_SHARED((8, _D), jnp.float32),
            pltpu.SemaphoreType.DMA,
        ],
    )
    def mk_mixed(starts_hbm, ends_hbm, table_hbm, zeros_hbm, mixed_hbm,
                 starts_v, ends_v, sp_table, sp_zeros, dsem):
        cid = lax.axis_index("c")
        sid = lax.axis_index("s")
        wid = cid * NS + sid
        nbase = pl.multiple_of(wid * NPW, NPW)
        pltpu.sync_copy(starts_hbm.at[pl.ds(nbase, NPW)],
                        starts_v.at[pl.ds(0, NPW)])
        pltpu.sync_copy(ends_hbm.at[pl.ds(nbase, NPW)],
                        ends_v.at[pl.ds(0, NPW)])

        TROWS = T // NS
        soff = pl.multiple_of(sid * TROWS, TROWS)
        pltpu.sync_copy(table_hbm.at[pl.ds(soff, TROWS)],
                        sp_table.at[pl.ds(soff, TROWS)])

        @pl.when(sid == 0)
        def _():
            pltpu.sync_copy(zeros_hbm.at[pl.ds(0, 8)], sp_zeros)

        plsc.subcore_barrier()

        def mk_branch(k, eff, mbase):
            t, r = divmod(k, 8)

            def br():
                if k >= 49:
                    nd = k - 48
                    pltpu.async_copy(sp_table.at[pl.ds(eff + 48, nd)],
                                     mixed_hbm.at[pl.ds(mbase, nd)], dsem)
                    if nd < 2:
                        pltpu.async_copy(
                            sp_zeros.at[pl.ds(0, 2 - nd)],
                            mixed_hbm.at[pl.ds(mbase + nd, 2 - nd)], dsem)
                elif r != 0:
                    pltpu.async_copy(sp_table.at[pl.ds(eff + 8 * t, r)],
                                     mixed_hbm.at[pl.ds(mbase, r)], dsem)
                    pltpu.async_copy(sp_zeros.at[pl.ds(0, 8 - r)],
                                     mixed_hbm.at[pl.ds(mbase + r, 8 - r)],
                                     dsem)
            return br

        def node_body(n, rows):
            sv = starts_v[pl.ds(n, L)]
            ev = ends_v[pl.ds(n, L)]
            s = sv[0]
            e = ev[0]
            meta = e < 0
            eff = jnp.where(meta, T - 1, s)
            cnt = jnp.where(meta, 1, e - s + 1)
            mbase = (nbase + n) * 8
            lax.switch(cnt - 1,
                       [mk_branch(k, eff, mbase) for k in range(1, _MW + 1)])
            r8 = cnt % 8
            add = jnp.where(cnt >= 49, 2, jnp.where(r8 == 0, 0, 8))
            return rows + add

        rows = lax.fori_loop(0, NPW, node_body, jnp.int32(0))

        unit = mixed_hbm.at[pl.ds(0, 1)]

        def drain(i, _):
            pltpu.make_async_copy(unit, unit, dsem).wait()
            return 0

        lax.fori_loop(0, rows, drain, 0)

    return mk_mixed


def _build_main(N, T):
    NC, NS, L = _mesh_info()
    NW = NC * NS
    NPW = N // NW
    CHN = 32
    NCH = NPW // CHN

    mesh = plsc.VectorSubcoreMesh(core_axis_name="c", subcore_axis_name="s")

    @functools.partial(
        pl.kernel,
        mesh=mesh,
        out_type=jax.ShapeDtypeStruct((N, _MW, _D), jnp.float32),
        scratch_types=[
            pltpu.VMEM((NPW + L,), jnp.int32),
            pltpu.VMEM((NPW + L,), jnp.int32),
            pltpu.VMEM_SHARED((T, _D), jnp.float32),
            pltpu.VMEM_SHARED((_ZR, _D), jnp.float32),
            pltpu.VMEM_SHARED((NS * CHN * 8, _D), jnp.float32),
            pltpu.SemaphoreType.DMA,
            pltpu.SemaphoreType.DMA,
            pltpu.SemaphoreType.DMA,
        ],
    )
    def main(starts_hbm, ends_hbm, shifted_hbm, zeros_hbm, mixed_hbm,
             out_hbm, starts_v, ends_v, sp_shift, sp_zeros, sp_mixed,
             dsem, dsem2, wsem):
        cid = lax.axis_index("c")
        sid = lax.axis_index("s")
        wid = cid * NS + sid
        nbase = pl.multiple_of(wid * NPW, NPW)
        pltpu.sync_copy(starts_hbm.at[pl.ds(nbase, NPW)],
                        starts_v.at[pl.ds(0, NPW)])
        pltpu.sync_copy(ends_hbm.at[pl.ds(nbase, NPW)],
                        ends_v.at[pl.ds(0, NPW)])

        @pl.when(sid == 0)
        def _():
            pltpu.sync_copy(zeros_hbm, sp_zeros)

        plsc.subcore_barrier()

        sbase = sid * CHN * 8
        TROWS = T // NS
        soff = pl.multiple_of(sid * TROWS, TROWS)

        def mk_branch1(k, n, j):
            t, r = divmod(k, 8)

            def br():
                if k >= 49:
                    pltpu.async_copy(sp_mixed.at[pl.ds(sbase + j * 8, 2)],
                                     out_hbm.at[n, pl.ds(48, 2)], wsem)
                    return
                if r != 0:
                    pltpu.async_copy(sp_mixed.at[pl.ds(sbase + j * 8, 8)],
                                     out_hbm.at[n, pl.ds(8 * t, 8)], wsem)
                z0 = 8 * t + (8 if r != 0 else 0)
                if z0 < 48:
                    pltpu.async_copy(sp_zeros.at[pl.ds(0, 48 - z0)],
                                     out_hbm.at[n, pl.ds(z0, 48 - z0)], dsem)
                pltpu.async_copy(sp_zeros.at[pl.ds(0, 2)],
                                 out_hbm.at[n, pl.ds(48, 2)], dsem)
            return br

        def chunk_body(c, rows1):
            cb = pl.multiple_of(c * CHN, CHN)
            pltpu.sync_copy(
                mixed_hbm.at[pl.ds(pl.multiple_of((nbase + cb) * 8, 8 * CHN),
                                   8 * CHN)],
                sp_mixed.at[pl.ds(pl.multiple_of(sbase, 8 * CHN), 8 * CHN)])

            def node_body(j, carry):
                rws, wrs = carry
                sv = starts_v[pl.ds(cb + j, L)]
                ev = ends_v[pl.ds(cb + j, L)]
                s = sv[0]
                e = ev[0]
                meta = e < 0
                cnt = jnp.where(meta, 1, e - s + 1)
                lax.switch(cnt - 1,
                           [mk_branch1(k, nbase + cb + j, j)
                            for k in range(1, _MW + 1)])
                r8 = cnt % 8
                wr = jnp.where(cnt >= 49, 2, jnp.where(r8 == 0, 0, 8))
                return rws + (_MW - 8 * (cnt // 8) - wr), wrs + wr

            rows1, wrows = lax.fori_loop(0, CHN, node_body,
                                         (rows1, jnp.int32(0)))

            unit1 = out_hbm.at[0, pl.ds(0, 1)]

            def drainw(i, _):
                pltpu.make_async_copy(unit1, unit1, wsem).wait()
                return 0

            lax.fori_loop(0, wrows, drainw, 0)
            return rows1

        rows1 = lax.fori_loop(0, NCH, chunk_body, jnp.int32(0))

        def mk_branch2(k, eff, n, rho):
            q = min(k // 8, 6)

            def br():
                if q > 0:
                    effa = pl.multiple_of(eff - rho, 8)
                    pltpu.async_copy(sp_shift.at[pl.ds(effa, 8 * q)],
                                     out_hbm.at[n, pl.ds(0, 8 * q)], dsem2)
            return br

        unit = out_hbm.at[0, pl.ds(0, 1)]

        def pass_body(rho, _):
            plsc.subcore_barrier()
            pltpu.sync_copy(shifted_hbm.at[rho, pl.ds(soff, TROWS)],
                            sp_shift.at[pl.ds(soff, TROWS)])
            plsc.subcore_barrier()

            def node_body(n, rows2):
                sv = starts_v[pl.ds(n, L)]
                ev = ends_v[pl.ds(n, L)]
                s = sv[0]
                e = ev[0]
                meta = e < 0
                eff = jnp.where(meta, T - 1, s)
                cnt = jnp.where(meta, 1, e - s + 1)
                mine = lax.rem(eff, 8) == rho

                @pl.when(mine)
                def _():
                    lax.switch(cnt - 1,
                               [mk_branch2(k, eff, nbase + n, rho)
                                for k in range(1, _MW + 1)])

                q = jnp.minimum(cnt // 8, 6)
                return rows2 + jnp.where(mine, 8 * q, 0)

            rows2 = lax.fori_loop(0, NPW, node_body, jnp.int32(0))

            def drain2(i, _):
                pltpu.make_async_copy(unit, unit, dsem2).wait()
                return 0

            lax.fori_loop(0, rows2, drain2, 0)
            return 0

        lax.fori_loop(0, 8, pass_body, 0)

        def drain1(i, _):
            pltpu.make_async_copy(unit, unit, dsem).wait()
            return 0

        lax.fori_loop(0, rows1, drain1, 0)

    return main


def kernel(batched_nodes, batched_bi_lstm_outputs):
    nodes0 = batched_nodes[0]
    H = batched_bi_lstm_outputs[0]
    N = nodes0.shape[0]
    T = H.shape[0]
    starts = nodes0[:, 0]
    ends = nodes0[:, 1]
    zeros = jnp.zeros((_ZR, _D), H.dtype)
    shifted = jnp.stack([jnp.roll(H, -r, axis=0) for r in range(8)])
    mixed = _build_mixed_tiles(N, T)(starts, ends, H, zeros)
    out = _build_main(N, T)(starts, ends, shifted, zeros, mixed)
    return out.reshape(1, N, _MW, _D)
